# baseline (speedup 1.0000x reference)
import functools

import jax
import jax.numpy as jnp
from jax.experimental import pallas as pl
from jax.experimental.pallas import tpu as pltpu

TINY_NUMBER = 1e-6
N_RAY, N_SAMPLE, N_SRC, DIM = 512, 64, 8, 64
H = DIM // 8
SB = 8


def _body(q_ref, k_ref, pos_ref, m_ref,
          wqs_ref, wkvs_ref, wp1_ref, bp1_ref, wp2s_ref, bp2_ref,
          ba1c_ref, wa2_ref, ba2_ref, wo_ref, bo_ref,
          x_ref, attn_ref, kstd_ref, nkstd_ref,
          vp_ref, a1k_ref, ph_ref):
    for s in range(SB):
        _sample(s, q_ref, k_ref, pos_ref, m_ref,
                wqs_ref, wkvs_ref, wp1_ref, bp1_ref, wp2s_ref, bp2_ref,
                ba1c_ref, wa2_ref, ba2_ref, wo_ref, bo_ref,
                x_ref, attn_ref, kstd_ref, nkstd_ref,
                vp_ref, a1k_ref, ph_ref)


def _sample(s, q_ref, k_ref, pos_ref, m_ref,
            wqs_ref, wkvs_ref, wp1_ref, bp1_ref, wp2s_ref, bp2_ref,
            ba1c_ref, wa2_ref, ba2_ref, wo_ref, bo_ref,
            x_ref, attn_ref, kstd_ref, nkstd_ref,
            vp_ref, a1k_ref, ph_ref):
    N = N_SRC
    D = DIM
    dot = functools.partial(jnp.dot, preferred_element_type=jnp.float32)

    qq = dot(wqs_ref[...], q_ref[s])
    qf = qq[:D]
    a1q = qq[D:]

    m = [m_ref[s, n:n + 1, :] for n in range(N)]
    cnt = m[0]
    for n in range(1, N):
        cnt = cnt + m[n]
    all_inv = cnt == 0.0
    cnt_eff = jnp.where(all_inv, float(N), cnt)
    inv_cnt = 1.0 / cnt_eff
    inv_cm1 = 1.0 / jnp.maximum(cnt_eff - 1.0, 1.0)
    w = [jnp.where(all_inv, 1.0, m[n]) for n in range(N)]

    swk = jnp.zeros_like(qf)
    swk2 = jnp.zeros_like(qf)
    sabs = jnp.zeros_like(qf)
    for n in range(N):
        kv = dot(wkvs_ref[...], k_ref[s, n])
        kf = kv[:D]
        vp_ref[n] = kv[D:2 * D]
        a1k_ref[n] = kv[2 * D:]
        kfw = kf * w[n]
        swk = swk + kfw
        swk2 = swk2 + kf * kfw
        sabs = sabs + jnp.abs(kfw)

    mean_k = swk * inv_cnt
    var = (swk2 - swk * mean_k) * inv_cm1
    single = cnt_eff == 1.0
    std = jnp.sqrt(jnp.where(single, 1.0, jnp.maximum(var, 0.0)))
    mean_abs = sabs * inv_cnt
    kstd_ref[s] = jnp.where(single, 0.0, std)
    nkstd_ref[s] = jnp.where(single, 0.0, std / (mean_abs + TINY_NUMBER))

    for n in range(N):
        ph_ref[n] = jnp.maximum(dot(wp1_ref[...], pos_ref[s, n]) + bp1_ref[...], 0.0)

    for n in range(N):
        pp = dot(wp2s_ref[...], ph_ref[n])
        vp_ref[n] = vp_ref[n] + pp[:D] + bp2_ref[...]
        a1k_ref[n] = jnp.maximum(a1k_ref[n] - a1q + pp[D:] + ba1c_ref[...], 0.0)

    for n in range(N):
        attn_ref[s, n] = dot(wa2_ref[...], a1k_ref[n]) + ba2_ref[...]

    neg = jnp.float32(-jnp.inf)
    lmax = jnp.where(w[0] == 0.0, neg, attn_ref[s, 0])
    for n in range(1, N):
        lmax = jnp.maximum(lmax, jnp.where(w[n] == 0.0, neg, attn_ref[s, n]))

    esum = jnp.zeros_like(qf)
    xe = jnp.zeros_like(qf)
    for n in range(N):
        e = jnp.where(w[n] == 0.0, 0.0, jnp.exp(attn_ref[s, n] - lmax))
        esum = esum + e
        xe = xe + vp_ref[n] * e
        attn_ref[s, n] = e
    inv_esum = 1.0 / esum

    for n in range(N):
        attn_ref[s, n] = attn_ref[s, n] * inv_esum

    x_ref[s] = dot(wo_ref[...], xe * inv_esum) + bo_ref[...]


def kernel(q, k, pos, mask, Wq, Wk, Wv, Wp1, bp1, Wp2, bp2, Wa1, ba1, Wa2, ba2, Wo, bo):
    R, S, N, D = N_RAY, N_SAMPLE, N_SRC, DIM
    f32 = jnp.float32

    qT = q.transpose(1, 2, 0)
    kT = k.transpose(1, 2, 3, 0)
    posT = pos.transpose(1, 2, 3, 0)
    mT = mask[..., 0].astype(f32).transpose(1, 2, 0)

    Wa1k = Wa1.T @ Wk.T
    Wa1q = Wa1.T @ Wq.T
    Wa1p = Wa1.T @ Wp2.T
    Wqs = jnp.concatenate([Wq.T, Wa1q], axis=0)
    Wkvs = jnp.concatenate([Wk.T, Wv.T, Wa1k], axis=0)
    Wp2s = jnp.concatenate([Wp2.T, Wa1p], axis=0)
    ba1c = Wa1.T @ bp2 + ba1

    bp1t = jnp.broadcast_to(bp1[:, None], (H, R))
    bp2t = jnp.broadcast_to(bp2[:, None], (D, R))
    ba1t = jnp.broadcast_to(ba1c[:, None], (H, R))
    ba2t = jnp.broadcast_to(ba2[:, None], (D, R))
    bot = jnp.broadcast_to(bo[:, None], (D, R))

    grid = (S // SB,)

    def wspec(wshape):
        return pl.BlockSpec(wshape, lambda s: (0, 0))

    out = pl.pallas_call(
        _body,
        grid=grid,
        in_specs=[
            pl.BlockSpec((SB, D, R), lambda s: (s, 0, 0)),
            pl.BlockSpec((SB, N, D, R), lambda s: (s, 0, 0, 0)),
            pl.BlockSpec((SB, N, 4, R), lambda s: (s, 0, 0, 0)),
            pl.BlockSpec((SB, N, R), lambda s: (s, 0, 0)),
            wspec((D + H, D)), wspec((2 * D + H, D)),
            wspec((H, 4)), wspec((H, R)), wspec((D + H, H)), wspec((D, R)),
            wspec((H, R)), wspec((D, H)), wspec((D, R)),
            wspec((D, D)), wspec((D, R)),
        ],
        out_specs=[
            pl.BlockSpec((SB, D, R), lambda s: (s, 0, 0)),
            pl.BlockSpec((SB, N, D, R), lambda s: (s, 0, 0, 0)),
            pl.BlockSpec((SB, D, R), lambda s: (s, 0, 0)),
            pl.BlockSpec((SB, D, R), lambda s: (s, 0, 0)),
        ],
        out_shape=[
            jax.ShapeDtypeStruct((S, D, R), f32),
            jax.ShapeDtypeStruct((S, N, D, R), f32),
            jax.ShapeDtypeStruct((S, D, R), f32),
            jax.ShapeDtypeStruct((S, D, R), f32),
        ],
        scratch_shapes=[
            pltpu.VMEM((N, D, R), f32),
            pltpu.VMEM((N, H, R), f32),
            pltpu.VMEM((N, H, R), f32),
        ],
    )(qT, kT, posT, mT, Wqs, Wkvs, Wp1.T, bp1t, Wp2s, bp2t,
      ba1t, Wa2.T, ba2t, Wo.T, bot)

    xT, attnT, kstdT, nkstdT = out
    return (xT.transpose(2, 0, 1), attnT.transpose(3, 0, 1, 2),
            kstdT.transpose(2, 0, 1), nkstdT.transpose(2, 0, 1))


# speedup vs baseline: 1.0244x; 1.0244x over previous
import functools

import jax
import jax.numpy as jnp
from jax.experimental import pallas as pl
from jax.experimental.pallas import tpu as pltpu

TINY_NUMBER = 1e-6
N_RAY, N_SAMPLE, N_SRC, DIM = 512, 64, 8, 64
H = DIM // 8
SB = 4


def _body(q_ref, k_ref, pos_ref, m_ref,
          wqs_ref, wkvs_ref, wp1_ref, bp1_ref, wp2s_ref, bp2_ref,
          ba1c_ref, wa2_ref, ba2_ref, wo_ref, bo_ref,
          x_ref, attn_ref, kstd_ref, nkstd_ref,
          vp_ref, a1k_ref, ph_ref):
    for s in range(SB):
        _sample(s, q_ref, k_ref, pos_ref, m_ref,
                wqs_ref, wkvs_ref, wp1_ref, bp1_ref, wp2s_ref, bp2_ref,
                ba1c_ref, wa2_ref, ba2_ref, wo_ref, bo_ref,
                x_ref, attn_ref, kstd_ref, nkstd_ref,
                vp_ref, a1k_ref, ph_ref)


def _sample(s, q_ref, k_ref, pos_ref, m_ref,
            wqs_ref, wkvs_ref, wp1_ref, bp1_ref, wp2s_ref, bp2_ref,
            ba1c_ref, wa2_ref, ba2_ref, wo_ref, bo_ref,
            x_ref, attn_ref, kstd_ref, nkstd_ref,
            vp_ref, a1k_ref, ph_ref):
    N = N_SRC
    D = DIM
    dot = functools.partial(jnp.dot, preferred_element_type=jnp.float32)

    qq = dot(wqs_ref[...], q_ref[s])
    qf = qq[:D]
    a1q = qq[D:]

    m = [m_ref[s, n:n + 1, :] for n in range(N)]
    cnt = m[0]
    for n in range(1, N):
        cnt = cnt + m[n]
    all_inv = cnt == 0.0
    cnt_eff = jnp.where(all_inv, float(N), cnt)
    inv_cnt = 1.0 / cnt_eff
    inv_cm1 = 1.0 / jnp.maximum(cnt_eff - 1.0, 1.0)
    w = [jnp.where(all_inv, 1.0, m[n]) for n in range(N)]

    swk = jnp.zeros_like(qf)
    swk2 = jnp.zeros_like(qf)
    sabs = jnp.zeros_like(qf)
    for n in range(N):
        kv = dot(wkvs_ref[...], k_ref[s, n])
        kf = kv[:D]
        vp_ref[n] = kv[D:2 * D]
        a1k_ref[n] = kv[2 * D:]
        kfw = kf * w[n]
        swk = swk + kfw
        swk2 = swk2 + kf * kfw
        sabs = sabs + jnp.abs(kfw)

    mean_k = swk * inv_cnt
    var = (swk2 - swk * mean_k) * inv_cm1
    single = cnt_eff == 1.0
    std = jnp.sqrt(jnp.where(single, 1.0, jnp.maximum(var, 0.0)))
    mean_abs = sabs * inv_cnt
    kstd_ref[s] = jnp.where(single, 0.0, std)
    nkstd_ref[s] = jnp.where(single, 0.0, std / (mean_abs + TINY_NUMBER))

    for n in range(N):
        ph_ref[n] = jnp.maximum(dot(wp1_ref[...], pos_ref[s, n]) + bp1_ref[...], 0.0)

    for n in range(N):
        pp = dot(wp2s_ref[...], ph_ref[n])
        vp_ref[n] = vp_ref[n] + pp[:D] + bp2_ref[...]
        a1k_ref[n] = jnp.maximum(a1k_ref[n] - a1q + pp[D:] + ba1c_ref[...], 0.0)

    for n in range(N):
        attn_ref[s, n] = dot(wa2_ref[...], a1k_ref[n]) + ba2_ref[...]

    neg = jnp.float32(-jnp.inf)
    lmax = jnp.where(w[0] == 0.0, neg, attn_ref[s, 0])
    for n in range(1, N):
        lmax = jnp.maximum(lmax, jnp.where(w[n] == 0.0, neg, attn_ref[s, n]))

    esum = jnp.zeros_like(qf)
    xe = jnp.zeros_like(qf)
    for n in range(N):
        e = jnp.where(w[n] == 0.0, 0.0, jnp.exp(attn_ref[s, n] - lmax))
        esum = esum + e
        xe = xe + vp_ref[n] * e
        attn_ref[s, n] = e
    inv_esum = 1.0 / esum

    for n in range(N):
        attn_ref[s, n] = attn_ref[s, n] * inv_esum

    x_ref[s] = dot(wo_ref[...], xe * inv_esum) + bo_ref[...]


def kernel(q, k, pos, mask, Wq, Wk, Wv, Wp1, bp1, Wp2, bp2, Wa1, ba1, Wa2, ba2, Wo, bo):
    R, S, N, D = N_RAY, N_SAMPLE, N_SRC, DIM
    f32 = jnp.float32

    qT = q.transpose(1, 2, 0)
    kT = k.transpose(1, 2, 3, 0)
    posT = pos.transpose(1, 2, 3, 0)
    mT = mask[..., 0].astype(f32).transpose(1, 2, 0)

    Wa1k = Wa1.T @ Wk.T
    Wa1q = Wa1.T @ Wq.T
    Wa1p = Wa1.T @ Wp2.T
    Wqs = jnp.concatenate([Wq.T, Wa1q], axis=0)
    Wkvs = jnp.concatenate([Wk.T, Wv.T, Wa1k], axis=0)
    Wp2s = jnp.concatenate([Wp2.T, Wa1p], axis=0)
    ba1c = Wa1.T @ bp2 + ba1

    bp1t = jnp.broadcast_to(bp1[:, None], (H, R))
    bp2t = jnp.broadcast_to(bp2[:, None], (D, R))
    ba1t = jnp.broadcast_to(ba1c[:, None], (H, R))
    ba2t = jnp.broadcast_to(ba2[:, None], (D, R))
    bot = jnp.broadcast_to(bo[:, None], (D, R))

    grid = (S // SB,)

    def wspec(wshape):
        return pl.BlockSpec(wshape, lambda s: (0, 0))

    out = pl.pallas_call(
        _body,
        grid=grid,
        in_specs=[
            pl.BlockSpec((SB, D, R), lambda s: (s, 0, 0)),
            pl.BlockSpec((SB, N, D, R), lambda s: (s, 0, 0, 0)),
            pl.BlockSpec((SB, N, 4, R), lambda s: (s, 0, 0, 0)),
            pl.BlockSpec((SB, N, R), lambda s: (s, 0, 0)),
            wspec((D + H, D)), wspec((2 * D + H, D)),
            wspec((H, 4)), wspec((H, R)), wspec((D + H, H)), wspec((D, R)),
            wspec((H, R)), wspec((D, H)), wspec((D, R)),
            wspec((D, D)), wspec((D, R)),
        ],
        out_specs=[
            pl.BlockSpec((SB, D, R), lambda s: (s, 0, 0)),
            pl.BlockSpec((SB, N, D, R), lambda s: (s, 0, 0, 0)),
            pl.BlockSpec((SB, D, R), lambda s: (s, 0, 0)),
            pl.BlockSpec((SB, D, R), lambda s: (s, 0, 0)),
        ],
        out_shape=[
            jax.ShapeDtypeStruct((S, D, R), f32),
            jax.ShapeDtypeStruct((S, N, D, R), f32),
            jax.ShapeDtypeStruct((S, D, R), f32),
            jax.ShapeDtypeStruct((S, D, R), f32),
        ],
        scratch_shapes=[
            pltpu.VMEM((N, D, R), f32),
            pltpu.VMEM((N, H, R), f32),
            pltpu.VMEM((N, H, R), f32),
        ],
        compiler_params=pltpu.CompilerParams(
            dimension_semantics=("parallel",),
        ),
    )(qT, kT, posT, mT, Wqs, Wkvs, Wp1.T, bp1t, Wp2s, bp2t,
      ba1t, Wa2.T, ba2t, Wo.T, bot)

    xT, attnT, kstdT, nkstdT = out
    return (xT.transpose(2, 0, 1), attnT.transpose(3, 0, 1, 2),
            kstdT.transpose(2, 0, 1), nkstdT.transpose(2, 0, 1))
